# baseline (speedup 1.0000x reference)
import functools

import jax
import jax.numpy as jnp
from jax import lax
from jax.experimental import pallas as pl
from jax.experimental.pallas import tpu as pltpu
from jax.experimental.pallas import tpu_sc as plsc

N = 10000
E = 160000
DIN = 256
DH = 512
DOUT = 256

IDXW = 128
EP_ROWS = 1280
EP = EP_ROWS * IDXW
NPAD = 10112
ZPT = 632

NC = 2
NS = 16

R = 1000
GRID = N // R

def _mesh():
    return plsc.VectorSubcoreMesh(core_axis_name="c", subcore_axis_name="s")


def _segsum_pair(t0, t1, srcp, dstp, zrows):
    rows_per_sub = EP_ROWS // NS
    NB = 2
    NHALF = 2
    HR = rows_per_sub // NHALF
    NGRP = HR // NB

    @functools.partial(
        pl.kernel,
        mesh=_mesh(),
        out_type=(
            jax.ShapeDtypeStruct((NPAD, IDXW), jnp.float32),
            jax.ShapeDtypeStruct((NPAD, IDXW), jnp.float32),
        ),
        scratch_types=[
            pltpu.VMEM((HR, IDXW), jnp.int32),
            pltpu.VMEM((HR, IDXW), jnp.int32),
        ]
        + [pltpu.VMEM((IDXW, IDXW), jnp.float32)] * NB
        + [pltpu.VMEM_SHARED((NPAD, IDXW), jnp.float32)]
        + [pltpu.SemaphoreType.DMA] * (2 * NB),
    )
    def body(t0_h, t1_h, src_h, dst_h, z_h, out0_h, out1_h,
             src_v, dst_v, *rest):
        rows_v = rest[:NB]
        acc_sh = rest[NB]
        gsem = rest[NB + 1:NB + 1 + NB]
        ssem = rest[NB + 1 + NB:]
        c = lax.axis_index("c")
        s = lax.axis_index("s")
        pltpu.sync_copy(z_h.at[pl.ds(s * ZPT, ZPT)],
                        acc_sh.at[pl.ds(s * ZPT, ZPT)])
        plsc.subcore_barrier()

        def gather_start(j, b):
            @pl.when(c == 0)
            def _():
                pltpu.async_copy(t0_h.at[src_v.at[j]], rows_v[b], gsem[b])

            @pl.when(c == 1)
            def _():
                pltpu.async_copy(t1_h.at[src_v.at[j]], rows_v[b], gsem[b])

        def gather_wait(j, b):
            @pl.when(c == 0)
            def _():
                pltpu.make_async_copy(t0_h.at[src_v.at[j]], rows_v[b],
                                      gsem[b]).wait()

            @pl.when(c == 1)
            def _():
                pltpu.make_async_copy(t1_h.at[src_v.at[j]], rows_v[b],
                                      gsem[b]).wait()

        for half in range(NHALF):
            row0 = s * rows_per_sub + half * HR
            pltpu.sync_copy(src_h.at[pl.ds(row0, HR)], src_v)
            pltpu.sync_copy(dst_h.at[pl.ds(row0, HR)], dst_v)
            for b in range(NB):
                gather_start(b, b)

            def group(g, carry):
                for b in range(NB):
                    j = g * NB + b
                    gather_wait(j, b)
                    pltpu.async_copy(rows_v[b], acc_sh.at[dst_v.at[j]],
                                     ssem[b], add=True)
                for b in range(NB):
                    j = g * NB + b
                    pltpu.make_async_copy(rows_v[b], acc_sh.at[dst_v.at[j]],
                                          ssem[b]).wait()

                    @pl.when(g < NGRP - 1)
                    def _():
                        gather_start((g + 1) * NB + b, b)
                return carry

            lax.fori_loop(0, NGRP, group, 0)
        plsc.subcore_barrier()

        @pl.when(c == 0)
        def _():
            pltpu.sync_copy(acc_sh.at[pl.ds(s * ZPT, ZPT)],
                            out0_h.at[pl.ds(s * ZPT, ZPT)])

        @pl.when(c == 1)
        def _():
            pltpu.sync_copy(acc_sh.at[pl.ds(s * ZPT, ZPT)],
                            out1_h.at[pl.ds(s * ZPT, ZPT)])

    return body(t0, t1, srcp, dstp, zrows)


def _degree_counts(dstp, zrows, ones128):
    rows_per_w = EP_ROWS // (NC * NS)

    @functools.partial(
        pl.kernel,
        mesh=_mesh(),
        out_type=(
            jax.ShapeDtypeStruct((NPAD, IDXW), jnp.float32),
            jax.ShapeDtypeStruct((NPAD, IDXW), jnp.float32),
        ),
        scratch_types=[
            pltpu.VMEM((rows_per_w, IDXW), jnp.int32),
            pltpu.VMEM((IDXW, IDXW), jnp.float32),
            pltpu.VMEM_SHARED((NPAD, IDXW), jnp.float32),
        ],
    )
    def body(dst_h, z_h, ones_h, out0_h, out1_h, dst_v, ones_v, acc_sh):
        c = lax.axis_index("c")
        s = lax.axis_index("s")
        wid = s * NC + c
        pltpu.sync_copy(z_h.at[pl.ds(s * ZPT, ZPT)],
                        acc_sh.at[pl.ds(s * ZPT, ZPT)])
        pltpu.sync_copy(dst_h.at[pl.ds(wid * rows_per_w, rows_per_w)], dst_v)
        pltpu.sync_copy(ones_h, ones_v)
        plsc.subcore_barrier()

        def step(j, carry):
            pltpu.sync_copy(ones_v, acc_sh.at[dst_v.at[j]], add=True)
            return carry

        lax.fori_loop(0, rows_per_w, step, 0)
        plsc.subcore_barrier()

        @pl.when(c == 0)
        def _():
            pltpu.sync_copy(acc_sh.at[pl.ds(s * ZPT, ZPT)],
                            out0_h.at[pl.ds(s * ZPT, ZPT)])

        @pl.when(c == 1)
        def _():
            pltpu.sync_copy(acc_sh.at[pl.ds(s * ZPT, ZPT)],
                            out1_h.at[pl.ds(s * ZPT, ZPT)])

    return body(dstp, zrows, ones128)


def _dotT(a, w):
    return lax.dot_general(a, w, (((1,), (1,)), ((), ())),
                           preferred_element_type=jnp.float32)


def _stats_update(stats_ref, y, i):
    @pl.when(i == 0)
    def _():
        stats_ref[...] = jnp.zeros_like(stats_ref)

    lane = lax.broadcasted_iota(jnp.int32, (1, 128), 1)
    s1 = jnp.sum(y)
    s2 = jnp.sum(y * y)
    stats_ref[...] += (jnp.where(lane == 0, s1, 0.0)
                       + jnp.where(lane == 1, s2, 0.0))


def _ln_scalars(stats_ref, count):
    s1 = stats_ref[0, 0]
    s2 = stats_ref[0, 1]
    m = s1 / count
    var = jnp.maximum(s2 / count - m * m, 0.0)
    inv = 1.0 / (jnp.sqrt(var) + 1e-5)
    return m, inv


def _inv_degree(c0_ref, c1_ref):
    cnt = (c0_ref[...] + c1_ref[...])[:, 0:1]
    return 1.0 / jnp.maximum(cnt, 1.0)


def _layerA(sums, cnts, feat, Wl, Wr, bl, dout):
    nsum = len(sums)
    nfeat = len(feat)

    def body(*refs):
        i = pl.program_id(0)
        sum_refs = refs[:nsum]
        c0_ref, c1_ref = refs[nsum:nsum + 2]
        feat_refs = refs[nsum + 2:nsum + 2 + nfeat]
        Wl_ref, Wr_ref, bl_ref = refs[nsum + 2 + nfeat:nsum + 5 + nfeat]
        y_ref, stats_ref = refs[nsum + 5 + nfeat:]
        invdeg = _inv_degree(c0_ref, c1_ref)
        agg = jnp.concatenate([r[...] for r in sum_refs], axis=1) * invdeg
        f = jnp.concatenate([r[...] for r in feat_refs], axis=1)
        y = _dotT(agg, Wl_ref[...]) + _dotT(f, Wr_ref[...]) + bl_ref[...]
        y_ref[...] = y
        _stats_update(stats_ref, y, i)

    din_a = sums[0].shape[1] * nsum
    din_f = feat[0].shape[1] * nfeat
    specs = (
        [pl.BlockSpec((R, sums[0].shape[1]), lambda i: (i, 0))] * nsum
        + [pl.BlockSpec((R, IDXW), lambda i: (i, 0))] * 2
        + [pl.BlockSpec((R, feat[0].shape[1]), lambda i: (i, 0))] * nfeat
        + [pl.BlockSpec((dout, din_a), lambda i: (0, 0)),
           pl.BlockSpec((dout, din_f), lambda i: (0, 0)),
           pl.BlockSpec((1, dout), lambda i: (0, 0))]
    )
    return pl.pallas_call(
        body,
        grid=(GRID,),
        in_specs=specs,
        out_specs=[pl.BlockSpec((R, dout), lambda i: (i, 0)),
                   pl.BlockSpec((1, 128), lambda i: (0, 0))],
        out_shape=[jax.ShapeDtypeStruct((N, dout), jnp.float32),
                   jax.ShapeDtypeStruct((1, 128), jnp.float32)],
    )(*sums, *cnts, *feat, Wl, Wr, bl)


def _prelu_ln(y, stats_ref, lnw_ref, lnb_ref, a_ref, count):
    m, inv = _ln_scalars(stats_ref, count)
    h = (y - m) * inv * lnw_ref[...] + lnb_ref[...]
    a = a_ref[0, 0]
    return jnp.where(h >= 0.0, h, a * h)


def _layerB0(y0, st0, lnw, lnb, a, x, Ws1):
    def body(y_ref, st_ref, lnw_ref, lnb_ref, a_ref, x_ref, Ws_ref,
             h1_ref, z0_ref, z1_ref, z2_ref, z3_ref):
        h1 = _prelu_ln(y_ref[...], st_ref, lnw_ref, lnb_ref, a_ref,
                       float(N * DH))
        h1_ref[...] = h1
        z = h1 + _dotT(x_ref[...], Ws_ref[...])
        z0_ref[...] = z[:, 0:128]
        z1_ref[...] = z[:, 128:256]
        z2_ref[...] = z[:, 256:384]
        z3_ref[...] = z[:, 384:512]

    return pl.pallas_call(
        body,
        grid=(GRID,),
        in_specs=[
            pl.BlockSpec((R, DH), lambda i: (i, 0)),
            pl.BlockSpec(memory_space=pltpu.SMEM),
            pl.BlockSpec((1, DH), lambda i: (0, 0)),
            pl.BlockSpec((1, DH), lambda i: (0, 0)),
            pl.BlockSpec(memory_space=pltpu.SMEM),
            pl.BlockSpec((R, DIN), lambda i: (i, 0)),
            pl.BlockSpec((DH, DIN), lambda i: (0, 0)),
        ],
        out_specs=[pl.BlockSpec((R, DH), lambda i: (i, 0))]
        + [pl.BlockSpec((R, 128), lambda i: (i, 0))] * 4,
        out_shape=[jax.ShapeDtypeStruct((N, DH), jnp.float32)]
        + [jax.ShapeDtypeStruct((N, 128), jnp.float32)] * 4,
    )(y0, st0, lnw, lnb, a, x, Ws1)


def _layerB1(y1, st1, lnw, lnb, a, h1, x, Ws2, Wl2):
    def body(y_ref, st_ref, lnw_ref, lnb_ref, a_ref, h1_ref, x_ref,
             Ws_ref, Wl2_ref, z2_ref, u0_ref, u1_ref):
        h2 = _prelu_ln(y_ref[...], st_ref, lnw_ref, lnb_ref, a_ref,
                       float(N * DH))
        z2 = h1_ref[...] + h2 + _dotT(x_ref[...], Ws_ref[...])
        z2_ref[...] = z2
        u2 = _dotT(z2, Wl2_ref[...])
        u0_ref[...] = u2[:, 0:128]
        u1_ref[...] = u2[:, 128:256]

    return pl.pallas_call(
        body,
        grid=(GRID,),
        in_specs=[
            pl.BlockSpec((R, DH), lambda i: (i, 0)),
            pl.BlockSpec(memory_space=pltpu.SMEM),
            pl.BlockSpec((1, DH), lambda i: (0, 0)),
            pl.BlockSpec((1, DH), lambda i: (0, 0)),
            pl.BlockSpec(memory_space=pltpu.SMEM),
            pl.BlockSpec((R, DH), lambda i: (i, 0)),
            pl.BlockSpec((R, DIN), lambda i: (i, 0)),
            pl.BlockSpec((DH, DIN), lambda i: (0, 0)),
            pl.BlockSpec((DOUT, DH), lambda i: (0, 0)),
        ],
        out_specs=[pl.BlockSpec((R, DH), lambda i: (i, 0)),
                   pl.BlockSpec((R, 128), lambda i: (i, 0)),
                   pl.BlockSpec((R, 128), lambda i: (i, 0))],
        out_shape=[jax.ShapeDtypeStruct((N, DH), jnp.float32),
                   jax.ShapeDtypeStruct((N, 128), jnp.float32),
                   jax.ShapeDtypeStruct((N, 128), jnp.float32)],
    )(y1, st1, lnw, lnb, a, h1, x, Ws2, Wl2)


def _layerA2(sums, cnts, z2, Wr2, bl2):
    def body(s0_ref, s1_ref, c0_ref, c1_ref, z2_ref, Wr_ref, bl_ref,
             y_ref, stats_ref):
        i = pl.program_id(0)
        invdeg = _inv_degree(c0_ref, c1_ref)
        agg = jnp.concatenate([s0_ref[...], s1_ref[...]], axis=1) * invdeg
        y = agg + _dotT(z2_ref[...], Wr_ref[...]) + bl_ref[...]
        y_ref[...] = y
        _stats_update(stats_ref, y, i)

    return pl.pallas_call(
        body,
        grid=(GRID,),
        in_specs=[
            pl.BlockSpec((R, 128), lambda i: (i, 0)),
            pl.BlockSpec((R, 128), lambda i: (i, 0)),
            pl.BlockSpec((R, IDXW), lambda i: (i, 0)),
            pl.BlockSpec((R, IDXW), lambda i: (i, 0)),
            pl.BlockSpec((R, DH), lambda i: (i, 0)),
            pl.BlockSpec((DOUT, DH), lambda i: (0, 0)),
            pl.BlockSpec((1, DOUT), lambda i: (0, 0)),
        ],
        out_specs=[pl.BlockSpec((R, DOUT), lambda i: (i, 0)),
                   pl.BlockSpec((1, 128), lambda i: (0, 0))],
        out_shape=[jax.ShapeDtypeStruct((N, DOUT), jnp.float32),
                   jax.ShapeDtypeStruct((1, 128), jnp.float32)],
    )(*sums, *cnts, z2, Wr2, bl2)


def _layerB2(y2, st2, lnw, lnb, a):
    def body(y_ref, st_ref, lnw_ref, lnb_ref, a_ref, out_ref):
        out_ref[...] = _prelu_ln(y_ref[...], st_ref, lnw_ref, lnb_ref,
                                 a_ref, float(N * DOUT))

    return pl.pallas_call(
        body,
        grid=(GRID,),
        in_specs=[
            pl.BlockSpec((R, DOUT), lambda i: (i, 0)),
            pl.BlockSpec(memory_space=pltpu.SMEM),
            pl.BlockSpec((1, DOUT), lambda i: (0, 0)),
            pl.BlockSpec((1, DOUT), lambda i: (0, 0)),
            pl.BlockSpec(memory_space=pltpu.SMEM),
        ],
        out_specs=pl.BlockSpec((R, DOUT), lambda i: (i, 0)),
        out_shape=jax.ShapeDtypeStruct((N, DOUT), jnp.float32),
    )(y2, st2, lnw, lnb, a)


def _segsum_pair_jnp(t0, t1, srcp, dstp, zrows):
    src = srcp.reshape(-1)[:E]
    dst = dstp.reshape(-1)[:E]
    o0 = jax.ops.segment_sum(t0[src], dst, num_segments=NPAD)
    o1 = jax.ops.segment_sum(t1[src], dst, num_segments=NPAD)
    return o0, o1


def _degree_counts_jnp(dstp, zrows16, ones16):
    dst = dstp.reshape(-1)[:E]
    cnt = jax.ops.segment_sum(jnp.ones((E,), jnp.float32), dst,
                              num_segments=NPAD)
    c = jnp.broadcast_to(cnt[:, None], (NPAD, 16)) * 0.5
    return c, c


def kernel(x, edge_index, Wl0, bl0, Wr0, Wl1, bl1, Wr1, Wl2, bl2, Wr2,
           Ws1, Ws2, ln0_w, ln0_b, ln1_w, ln1_b, ln2_w, ln2_b, a0, a1, a2):
    src = edge_index[0]
    dst = edge_index[1]
    order = jnp.argsort(src)
    src = src[order]
    dst = dst[order]
    pad = EP - E
    srcp = jnp.concatenate(
        [src, jnp.zeros((pad,), jnp.int32)]).reshape(EP_ROWS, IDXW)
    dstp = jnp.concatenate(
        [dst, jnp.full((pad,), N, jnp.int32)]).reshape(EP_ROWS, IDXW)
    z128 = jnp.zeros((NPAD, 128), jnp.float32)
    ones128 = jnp.ones((IDXW, IDXW), jnp.float32)

    c0, c1 = _degree_counts(dstp, z128, ones128)
    cnts = (c0, c1)

    x0 = x[:, :128]
    x1 = x[:, 128:]
    s0a, s0b = _segsum_pair(x0, x1, srcp, dstp, z128)
    y0, st0 = _layerA((s0a, s0b), cnts, (x,), Wl0, Wr0,
                      bl0.reshape(1, -1), DH)
    h1, z1a, z1b, z1c, z1d = _layerB0(
        y0, st0, ln0_w.reshape(1, -1), ln0_b.reshape(1, -1),
        jnp.reshape(a0, (1, 1)), x, Ws1)

    s1a, s1b = _segsum_pair(z1a, z1b, srcp, dstp, z128)
    s1c, s1d = _segsum_pair(z1c, z1d, srcp, dstp, z128)
    y1, st1 = _layerA((s1a, s1b, s1c, s1d), cnts, (z1a, z1b, z1c, z1d),
                      Wl1, Wr1, bl1.reshape(1, -1), DH)
    z2, u2a, u2b = _layerB1(
        y1, st1, ln1_w.reshape(1, -1), ln1_b.reshape(1, -1),
        jnp.reshape(a1, (1, 1)), h1, x, Ws2, Wl2)

    s2a, s2b = _segsum_pair(u2a, u2b, srcp, dstp, z128)
    y2, st2 = _layerA2((s2a, s2b), cnts, z2, Wr2, bl2.reshape(1, -1))
    ret = _layerB2(y2, st2, ln2_w.reshape(1, -1), ln2_b.reshape(1, -1),
                   jnp.reshape(a2, (1, 1)))
    return ret


# speedup vs baseline: 1.2415x; 1.2415x over previous
import functools

import jax
import jax.numpy as jnp
from jax import lax
from jax.experimental import pallas as pl
from jax.experimental.pallas import tpu as pltpu
from jax.experimental.pallas import tpu_sc as plsc

N = 10000
E = 160000
DIN = 256
DH = 512
DOUT = 256

IDXW = 128
EP_ROWS = 1280
EP = EP_ROWS * IDXW
NPAD = 10112
ZPT = 632

NC = 2
NS = 16

R = 1000
GRID = N // R

def _mesh():
    return plsc.VectorSubcoreMesh(core_axis_name="c", subcore_axis_name="s")


def _segsum_pair(t0, t1, srcp, dstp, zrows):
    rows_per_sub = EP_ROWS // NS
    NB = 2
    NHALF = 2
    HR = rows_per_sub // NHALF
    NGRP = HR // NB

    @functools.partial(
        pl.kernel,
        mesh=_mesh(),
        out_type=(
            jax.ShapeDtypeStruct((NPAD, IDXW), jnp.float32),
            jax.ShapeDtypeStruct((NPAD, IDXW), jnp.float32),
        ),
        scratch_types=[
            pltpu.VMEM((HR, IDXW), jnp.int32),
            pltpu.VMEM((HR, IDXW), jnp.int32),
        ]
        + [pltpu.VMEM((IDXW, IDXW), jnp.float32)] * NB
        + [pltpu.VMEM_SHARED((NPAD, IDXW), jnp.float32)]
        + [pltpu.SemaphoreType.DMA] * (2 * NB),
    )
    def body(t0_h, t1_h, src_h, dst_h, z_h, out0_h, out1_h,
             src_v, dst_v, *rest):
        rows_v = rest[:NB]
        acc_sh = rest[NB]
        gsem = rest[NB + 1:NB + 1 + NB]
        ssem = rest[NB + 1 + NB:]
        c = lax.axis_index("c")
        s = lax.axis_index("s")
        pltpu.sync_copy(z_h.at[pl.ds(s * ZPT, ZPT)],
                        acc_sh.at[pl.ds(s * ZPT, ZPT)])
        plsc.subcore_barrier()

        def gather_start(j, b):
            @pl.when(c == 0)
            def _():
                pltpu.async_copy(t0_h.at[src_v.at[j]], rows_v[b], gsem[b])

            @pl.when(c == 1)
            def _():
                pltpu.async_copy(t1_h.at[src_v.at[j]], rows_v[b], gsem[b])

        def gather_wait(j, b):
            @pl.when(c == 0)
            def _():
                pltpu.make_async_copy(t0_h.at[src_v.at[j]], rows_v[b],
                                      gsem[b]).wait()

            @pl.when(c == 1)
            def _():
                pltpu.make_async_copy(t1_h.at[src_v.at[j]], rows_v[b],
                                      gsem[b]).wait()

        for half in range(NHALF):
            row0 = s * rows_per_sub + half * HR
            pltpu.sync_copy(src_h.at[pl.ds(row0, HR)], src_v)
            pltpu.sync_copy(dst_h.at[pl.ds(row0, HR)], dst_v)
            for b in range(NB):
                gather_start(b, b)

            def group(g, carry):
                for b in range(NB):
                    j = g * NB + b
                    gather_wait(j, b)
                    pltpu.async_copy(rows_v[b], acc_sh.at[dst_v.at[j]],
                                     ssem[b], add=True)
                for b in range(NB):
                    j = g * NB + b
                    pltpu.make_async_copy(rows_v[b], acc_sh.at[dst_v.at[j]],
                                          ssem[b]).wait()

                    @pl.when(g < NGRP - 1)
                    def _():
                        gather_start((g + 1) * NB + b, b)
                return carry

            lax.fori_loop(0, NGRP, group, 0)
        plsc.subcore_barrier()

        @pl.when(c == 0)
        def _():
            pltpu.sync_copy(acc_sh.at[pl.ds(s * ZPT, ZPT)],
                            out0_h.at[pl.ds(s * ZPT, ZPT)])

        @pl.when(c == 1)
        def _():
            pltpu.sync_copy(acc_sh.at[pl.ds(s * ZPT, ZPT)],
                            out1_h.at[pl.ds(s * ZPT, ZPT)])

    return body(t0, t1, srcp, dstp, zrows)


def _degree_counts(dstp, zrows, ones128):
    rows_per_w = EP_ROWS // (NC * NS)

    @functools.partial(
        pl.kernel,
        mesh=_mesh(),
        out_type=(
            jax.ShapeDtypeStruct((NPAD, IDXW), jnp.float32),
            jax.ShapeDtypeStruct((NPAD, IDXW), jnp.float32),
        ),
        scratch_types=[
            pltpu.VMEM((rows_per_w, IDXW), jnp.int32),
            pltpu.VMEM((IDXW, IDXW), jnp.float32),
            pltpu.VMEM_SHARED((NPAD, IDXW), jnp.float32),
        ],
    )
    def body(dst_h, z_h, ones_h, out0_h, out1_h, dst_v, ones_v, acc_sh):
        c = lax.axis_index("c")
        s = lax.axis_index("s")
        wid = s * NC + c
        pltpu.sync_copy(z_h.at[pl.ds(s * ZPT, ZPT)],
                        acc_sh.at[pl.ds(s * ZPT, ZPT)])
        pltpu.sync_copy(dst_h.at[pl.ds(wid * rows_per_w, rows_per_w)], dst_v)
        pltpu.sync_copy(ones_h, ones_v)
        plsc.subcore_barrier()

        def step(j, carry):
            pltpu.sync_copy(ones_v, acc_sh.at[dst_v.at[j]], add=True)
            return carry

        lax.fori_loop(0, rows_per_w, step, 0)
        plsc.subcore_barrier()

        @pl.when(c == 0)
        def _():
            pltpu.sync_copy(acc_sh.at[pl.ds(s * ZPT, ZPT)],
                            out0_h.at[pl.ds(s * ZPT, ZPT)])

        @pl.when(c == 1)
        def _():
            pltpu.sync_copy(acc_sh.at[pl.ds(s * ZPT, ZPT)],
                            out1_h.at[pl.ds(s * ZPT, ZPT)])

    return body(dstp, zrows, ones128)


def _dotT(a, w):
    return lax.dot_general(a, w, (((1,), (1,)), ((), ())),
                           preferred_element_type=jnp.float32)


def _stats_update(stats_ref, y, i):
    @pl.when(i == 0)
    def _():
        stats_ref[...] = jnp.zeros_like(stats_ref)

    lane = lax.broadcasted_iota(jnp.int32, (1, 128), 1)
    s1 = jnp.sum(y)
    s2 = jnp.sum(y * y)
    stats_ref[...] += (jnp.where(lane == 0, s1, 0.0)
                       + jnp.where(lane == 1, s2, 0.0))


def _ln_scalars(stats_ref, count):
    s1 = stats_ref[0, 0]
    s2 = stats_ref[0, 1]
    m = s1 / count
    var = jnp.maximum(s2 / count - m * m, 0.0)
    inv = 1.0 / (jnp.sqrt(var) + 1e-5)
    return m, inv


def _inv_degree(c0_ref, c1_ref):
    cnt = (c0_ref[...] + c1_ref[...])[:, 0:1]
    return 1.0 / jnp.maximum(cnt, 1.0)


def _layerA(sums, cnts, feat, Wl, Wr, bl, dout):
    nsum = len(sums)
    nfeat = len(feat)

    def body(*refs):
        i = pl.program_id(0)
        sum_refs = refs[:nsum]
        c0_ref, c1_ref = refs[nsum:nsum + 2]
        feat_refs = refs[nsum + 2:nsum + 2 + nfeat]
        Wl_ref, Wr_ref, bl_ref = refs[nsum + 2 + nfeat:nsum + 5 + nfeat]
        y_ref, stats_ref = refs[nsum + 5 + nfeat:]
        invdeg = _inv_degree(c0_ref, c1_ref)
        agg = jnp.concatenate([r[...] for r in sum_refs], axis=1) * invdeg
        f = jnp.concatenate([r[...] for r in feat_refs], axis=1)
        y = _dotT(agg, Wl_ref[...]) + _dotT(f, Wr_ref[...]) + bl_ref[...]
        y_ref[...] = y
        _stats_update(stats_ref, y, i)

    din_a = sums[0].shape[1] * nsum
    din_f = feat[0].shape[1] * nfeat
    specs = (
        [pl.BlockSpec((R, sums[0].shape[1]), lambda i: (i, 0))] * nsum
        + [pl.BlockSpec((R, IDXW), lambda i: (i, 0))] * 2
        + [pl.BlockSpec((R, feat[0].shape[1]), lambda i: (i, 0))] * nfeat
        + [pl.BlockSpec((dout, din_a), lambda i: (0, 0)),
           pl.BlockSpec((dout, din_f), lambda i: (0, 0)),
           pl.BlockSpec((1, dout), lambda i: (0, 0))]
    )
    return pl.pallas_call(
        body,
        grid=(GRID,),
        in_specs=specs,
        out_specs=[pl.BlockSpec((R, dout), lambda i: (i, 0)),
                   pl.BlockSpec((1, 128), lambda i: (0, 0))],
        out_shape=[jax.ShapeDtypeStruct((N, dout), jnp.float32),
                   jax.ShapeDtypeStruct((1, 128), jnp.float32)],
    )(*sums, *cnts, *feat, Wl, Wr, bl)


def _prelu_ln(y, stats_ref, lnw_ref, lnb_ref, a_ref, count):
    m, inv = _ln_scalars(stats_ref, count)
    h = (y - m) * inv * lnw_ref[...] + lnb_ref[...]
    a = a_ref[0, 0]
    return jnp.where(h >= 0.0, h, a * h)


def _layerB0(y0, st0, lnw, lnb, a, x, Ws1):
    def body(y_ref, st_ref, lnw_ref, lnb_ref, a_ref, x_ref, Ws_ref,
             h1_ref, z0_ref, z1_ref, z2_ref, z3_ref):
        h1 = _prelu_ln(y_ref[...], st_ref, lnw_ref, lnb_ref, a_ref,
                       float(N * DH))
        h1_ref[...] = h1
        z = h1 + _dotT(x_ref[...], Ws_ref[...])
        z0_ref[...] = z[:, 0:128]
        z1_ref[...] = z[:, 128:256]
        z2_ref[...] = z[:, 256:384]
        z3_ref[...] = z[:, 384:512]

    return pl.pallas_call(
        body,
        grid=(GRID,),
        in_specs=[
            pl.BlockSpec((R, DH), lambda i: (i, 0)),
            pl.BlockSpec(memory_space=pltpu.SMEM),
            pl.BlockSpec((1, DH), lambda i: (0, 0)),
            pl.BlockSpec((1, DH), lambda i: (0, 0)),
            pl.BlockSpec(memory_space=pltpu.SMEM),
            pl.BlockSpec((R, DIN), lambda i: (i, 0)),
            pl.BlockSpec((DH, DIN), lambda i: (0, 0)),
        ],
        out_specs=[pl.BlockSpec((R, DH), lambda i: (i, 0))]
        + [pl.BlockSpec((R, 128), lambda i: (i, 0))] * 4,
        out_shape=[jax.ShapeDtypeStruct((N, DH), jnp.float32)]
        + [jax.ShapeDtypeStruct((N, 128), jnp.float32)] * 4,
    )(y0, st0, lnw, lnb, a, x, Ws1)


def _layerB1(y1, st1, lnw, lnb, a, h1, x, Ws2, Wl2):
    def body(y_ref, st_ref, lnw_ref, lnb_ref, a_ref, h1_ref, x_ref,
             Ws_ref, Wl2_ref, z2_ref, u0_ref, u1_ref):
        h2 = _prelu_ln(y_ref[...], st_ref, lnw_ref, lnb_ref, a_ref,
                       float(N * DH))
        z2 = h1_ref[...] + h2 + _dotT(x_ref[...], Ws_ref[...])
        z2_ref[...] = z2
        u2 = _dotT(z2, Wl2_ref[...])
        u0_ref[...] = u2[:, 0:128]
        u1_ref[...] = u2[:, 128:256]

    return pl.pallas_call(
        body,
        grid=(GRID,),
        in_specs=[
            pl.BlockSpec((R, DH), lambda i: (i, 0)),
            pl.BlockSpec(memory_space=pltpu.SMEM),
            pl.BlockSpec((1, DH), lambda i: (0, 0)),
            pl.BlockSpec((1, DH), lambda i: (0, 0)),
            pl.BlockSpec(memory_space=pltpu.SMEM),
            pl.BlockSpec((R, DH), lambda i: (i, 0)),
            pl.BlockSpec((R, DIN), lambda i: (i, 0)),
            pl.BlockSpec((DH, DIN), lambda i: (0, 0)),
            pl.BlockSpec((DOUT, DH), lambda i: (0, 0)),
        ],
        out_specs=[pl.BlockSpec((R, DH), lambda i: (i, 0)),
                   pl.BlockSpec((R, 128), lambda i: (i, 0)),
                   pl.BlockSpec((R, 128), lambda i: (i, 0))],
        out_shape=[jax.ShapeDtypeStruct((N, DH), jnp.float32),
                   jax.ShapeDtypeStruct((N, 128), jnp.float32),
                   jax.ShapeDtypeStruct((N, 128), jnp.float32)],
    )(y1, st1, lnw, lnb, a, h1, x, Ws2, Wl2)


def _layerA2(sums, cnts, z2, Wr2, bl2):
    def body(s0_ref, s1_ref, c0_ref, c1_ref, z2_ref, Wr_ref, bl_ref,
             y_ref, stats_ref):
        i = pl.program_id(0)
        invdeg = _inv_degree(c0_ref, c1_ref)
        agg = jnp.concatenate([s0_ref[...], s1_ref[...]], axis=1) * invdeg
        y = agg + _dotT(z2_ref[...], Wr_ref[...]) + bl_ref[...]
        y_ref[...] = y
        _stats_update(stats_ref, y, i)

    return pl.pallas_call(
        body,
        grid=(GRID,),
        in_specs=[
            pl.BlockSpec((R, 128), lambda i: (i, 0)),
            pl.BlockSpec((R, 128), lambda i: (i, 0)),
            pl.BlockSpec((R, IDXW), lambda i: (i, 0)),
            pl.BlockSpec((R, IDXW), lambda i: (i, 0)),
            pl.BlockSpec((R, DH), lambda i: (i, 0)),
            pl.BlockSpec((DOUT, DH), lambda i: (0, 0)),
            pl.BlockSpec((1, DOUT), lambda i: (0, 0)),
        ],
        out_specs=[pl.BlockSpec((R, DOUT), lambda i: (i, 0)),
                   pl.BlockSpec((1, 128), lambda i: (0, 0))],
        out_shape=[jax.ShapeDtypeStruct((N, DOUT), jnp.float32),
                   jax.ShapeDtypeStruct((1, 128), jnp.float32)],
    )(*sums, *cnts, z2, Wr2, bl2)


def _layerB2(y2, st2, lnw, lnb, a):
    def body(y_ref, st_ref, lnw_ref, lnb_ref, a_ref, out_ref):
        out_ref[...] = _prelu_ln(y_ref[...], st_ref, lnw_ref, lnb_ref,
                                 a_ref, float(N * DOUT))

    return pl.pallas_call(
        body,
        grid=(GRID,),
        in_specs=[
            pl.BlockSpec((R, DOUT), lambda i: (i, 0)),
            pl.BlockSpec(memory_space=pltpu.SMEM),
            pl.BlockSpec((1, DOUT), lambda i: (0, 0)),
            pl.BlockSpec((1, DOUT), lambda i: (0, 0)),
            pl.BlockSpec(memory_space=pltpu.SMEM),
        ],
        out_specs=pl.BlockSpec((R, DOUT), lambda i: (i, 0)),
        out_shape=jax.ShapeDtypeStruct((N, DOUT), jnp.float32),
    )(y2, st2, lnw, lnb, a)


def _segsum_pair_jnp(t0, t1, srcp, dstp, zrows):
    src = srcp.reshape(-1)[:E]
    dst = dstp.reshape(-1)[:E]
    o0 = jax.ops.segment_sum(t0[src], dst, num_segments=NPAD)
    o1 = jax.ops.segment_sum(t1[src], dst, num_segments=NPAD)
    return o0, o1


def _degree_counts_jnp(dstp, zrows16, ones16):
    dst = dstp.reshape(-1)[:E]
    cnt = jax.ops.segment_sum(jnp.ones((E,), jnp.float32), dst,
                              num_segments=NPAD)
    c = jnp.broadcast_to(cnt[:, None], (NPAD, 16)) * 0.5
    return c, c


def kernel(x, edge_index, Wl0, bl0, Wr0, Wl1, bl1, Wr1, Wl2, bl2, Wr2,
           Ws1, Ws2, ln0_w, ln0_b, ln1_w, ln1_b, ln2_w, ln2_b, a0, a1, a2):
    src = edge_index[0]
    dst = edge_index[1]
    pad = EP - E
    srcp = jnp.concatenate(
        [src, jnp.zeros((pad,), jnp.int32)]).reshape(EP_ROWS, IDXW)
    dstp = jnp.concatenate(
        [dst, jnp.full((pad,), N, jnp.int32)]).reshape(EP_ROWS, IDXW)
    z128 = jnp.zeros((NPAD, 128), jnp.float32)
    ones128 = jnp.ones((IDXW, IDXW), jnp.float32)

    c0, c1 = _degree_counts(dstp, z128, ones128)
    cnts = (c0, c1)

    x0 = x[:, :128]
    x1 = x[:, 128:]
    s0a, s0b = _segsum_pair(x0, x1, srcp, dstp, z128)
    y0, st0 = _layerA((s0a, s0b), cnts, (x,), Wl0, Wr0,
                      bl0.reshape(1, -1), DH)
    h1, z1a, z1b, z1c, z1d = _layerB0(
        y0, st0, ln0_w.reshape(1, -1), ln0_b.reshape(1, -1),
        jnp.reshape(a0, (1, 1)), x, Ws1)

    s1a, s1b = _segsum_pair(z1a, z1b, srcp, dstp, z128)
    s1c, s1d = _segsum_pair(z1c, z1d, srcp, dstp, z128)
    y1, st1 = _layerA((s1a, s1b, s1c, s1d), cnts, (z1a, z1b, z1c, z1d),
                      Wl1, Wr1, bl1.reshape(1, -1), DH)
    z2, u2a, u2b = _layerB1(
        y1, st1, ln1_w.reshape(1, -1), ln1_b.reshape(1, -1),
        jnp.reshape(a1, (1, 1)), h1, x, Ws2, Wl2)

    s2a, s2b = _segsum_pair(u2a, u2b, srcp, dstp, z128)
    y2, st2 = _layerA2((s2a, s2b), cnts, z2, Wr2, bl2.reshape(1, -1))
    ret = _layerB2(y2, st2, ln2_w.reshape(1, -1), ln2_b.reshape(1, -1),
                   jnp.reshape(a2, (1, 1)))
    return ret


# speedup vs baseline: 1.2589x; 1.0140x over previous
import functools

import jax
import jax.numpy as jnp
from jax import lax
from jax.experimental import pallas as pl
from jax.experimental.pallas import tpu as pltpu
from jax.experimental.pallas import tpu_sc as plsc

N = 10000
E = 160000
DIN = 256
DH = 512
DOUT = 256

IDXW = 128
EP_ROWS = 1280
EP = EP_ROWS * IDXW
NPAD = 10112
ZPT = 632

NC = 2
NS = 16

R = 1000
GRID = N // R

def _mesh():
    return plsc.VectorSubcoreMesh(core_axis_name="c", subcore_axis_name="s")


def _segsum_pair(t0, t1, srcp, dstp, zrows):
    rows_per_sub = EP_ROWS // NS
    NB = 2
    NHALF = 2
    HR = rows_per_sub // NHALF
    NGRP = HR // NB

    @functools.partial(
        pl.kernel,
        mesh=_mesh(),
        out_type=(
            jax.ShapeDtypeStruct((NPAD, IDXW), jnp.float32),
            jax.ShapeDtypeStruct((NPAD, IDXW), jnp.float32),
        ),
        scratch_types=[
            pltpu.VMEM((HR, IDXW), jnp.int32),
            pltpu.VMEM((HR, IDXW), jnp.int32),
        ]
        + [pltpu.VMEM((IDXW, IDXW), jnp.float32)] * NB
        + [pltpu.VMEM_SHARED((NPAD, IDXW), jnp.float32)]
        + [pltpu.SemaphoreType.DMA] * (2 * NB),
    )
    def body(t0_h, t1_h, src_h, dst_h, z_h, out0_h, out1_h,
             src_v, dst_v, *rest):
        rows_v = rest[:NB]
        acc_sh = rest[NB]
        gsem = rest[NB + 1:NB + 1 + NB]
        ssem = rest[NB + 1 + NB:]
        c = lax.axis_index("c")
        s = lax.axis_index("s")
        pltpu.sync_copy(z_h.at[pl.ds(s * ZPT, ZPT)],
                        acc_sh.at[pl.ds(s * ZPT, ZPT)])
        plsc.subcore_barrier()

        def gather_start(j, b):
            @pl.when(c == 0)
            def _():
                pltpu.async_copy(t0_h.at[src_v.at[j]], rows_v[b], gsem[b])

            @pl.when(c == 1)
            def _():
                pltpu.async_copy(t1_h.at[src_v.at[j]], rows_v[b], gsem[b])

        def gather_wait(j, b):
            @pl.when(c == 0)
            def _():
                pltpu.make_async_copy(t0_h.at[src_v.at[j]], rows_v[b],
                                      gsem[b]).wait()

            @pl.when(c == 1)
            def _():
                pltpu.make_async_copy(t1_h.at[src_v.at[j]], rows_v[b],
                                      gsem[b]).wait()

        for half in range(NHALF):
            row0 = s * rows_per_sub + half * HR
            pltpu.sync_copy(src_h.at[pl.ds(row0, HR)], src_v)
            pltpu.sync_copy(dst_h.at[pl.ds(row0, HR)], dst_v)
            for b in range(NB):
                gather_start(b, b)

            def group(g, carry):
                for b in range(NB):
                    j = g * NB + b
                    gather_wait(j, b)
                    pltpu.async_copy(rows_v[b], acc_sh.at[dst_v.at[j]],
                                     ssem[b], add=True)
                for b in range(NB):
                    j = g * NB + b
                    pltpu.make_async_copy(rows_v[b], acc_sh.at[dst_v.at[j]],
                                          ssem[b]).wait()

                    @pl.when(g < NGRP - 1)
                    def _():
                        gather_start((g + 1) * NB + b, b)
                return carry

            lax.fori_loop(0, NGRP, group, 0)
        plsc.subcore_barrier()

        @pl.when(c == 0)
        def _():
            pltpu.sync_copy(acc_sh.at[pl.ds(s * ZPT, ZPT)],
                            out0_h.at[pl.ds(s * ZPT, ZPT)])

        @pl.when(c == 1)
        def _():
            pltpu.sync_copy(acc_sh.at[pl.ds(s * ZPT, ZPT)],
                            out1_h.at[pl.ds(s * ZPT, ZPT)])

    return body(t0, t1, srcp, dstp, zrows)


def _degree_counts(dstp, zrows, ones128):
    rows_per_w = EP_ROWS // (NC * NS)

    @functools.partial(
        pl.kernel,
        mesh=_mesh(),
        out_type=(
            jax.ShapeDtypeStruct((NPAD, IDXW), jnp.float32),
            jax.ShapeDtypeStruct((NPAD, IDXW), jnp.float32),
        ),
        scratch_types=[
            pltpu.VMEM((rows_per_w, IDXW), jnp.int32),
            pltpu.VMEM((IDXW, IDXW), jnp.float32),
            pltpu.VMEM_SHARED((NPAD, IDXW), jnp.float32),
        ],
    )
    def body(dst_h, z_h, ones_h, out0_h, out1_h, dst_v, ones_v, acc_sh):
        c = lax.axis_index("c")
        s = lax.axis_index("s")
        wid = s * NC + c
        pltpu.sync_copy(z_h.at[pl.ds(s * ZPT, ZPT)],
                        acc_sh.at[pl.ds(s * ZPT, ZPT)])
        pltpu.sync_copy(dst_h.at[pl.ds(wid * rows_per_w, rows_per_w)], dst_v)
        pltpu.sync_copy(ones_h, ones_v)
        plsc.subcore_barrier()

        def step(j, carry):
            pltpu.sync_copy(ones_v, acc_sh.at[dst_v.at[j]], add=True)
            return carry

        lax.fori_loop(0, rows_per_w, step, 0)
        plsc.subcore_barrier()

        @pl.when(c == 0)
        def _():
            pltpu.sync_copy(acc_sh.at[pl.ds(s * ZPT, ZPT)],
                            out0_h.at[pl.ds(s * ZPT, ZPT)])

        @pl.when(c == 1)
        def _():
            pltpu.sync_copy(acc_sh.at[pl.ds(s * ZPT, ZPT)],
                            out1_h.at[pl.ds(s * ZPT, ZPT)])

    return body(dstp, zrows, ones128)


def _dotT(a, w):
    return lax.dot_general(a, w, (((1,), (1,)), ((), ())),
                           preferred_element_type=jnp.float32)


def _stats_update(stats_ref, y, i):
    @pl.when(i == 0)
    def _():
        stats_ref[...] = jnp.zeros_like(stats_ref)

    lane = lax.broadcasted_iota(jnp.int32, (1, 128), 1)
    s1 = jnp.sum(y)
    s2 = jnp.sum(y * y)
    stats_ref[...] += (jnp.where(lane == 0, s1, 0.0)
                       + jnp.where(lane == 1, s2, 0.0))


def _ln_scalars(stats_ref, count):
    s1 = stats_ref[0, 0]
    s2 = stats_ref[0, 1]
    m = s1 / count
    var = jnp.maximum(s2 / count - m * m, 0.0)
    inv = 1.0 / (jnp.sqrt(var) + 1e-5)
    return m, inv


def _inv_degree(c0_ref, c1_ref):
    cnt = (c0_ref[...] + c1_ref[...])[:, 0:1]
    return 1.0 / jnp.maximum(cnt, 1.0)


def _ph0(i):
    return (jnp.where(i < GRID, i, 0), 0)


def _ph1(i):
    return (jnp.where(i < GRID, 0, i - GRID), 0)


def _both(i):
    return (lax.rem(i, GRID), 0)


def _const(i):
    return (0, 0)


def _smem_spec():
    return pl.BlockSpec(memory_space=pltpu.SMEM)


def _stats_add(stats_ref, y, i):
    @pl.when(i == 0)
    def _():
        stats_ref[0] = 0.0
        stats_ref[1] = 0.0

    stats_ref[0] += jnp.sum(y)
    stats_ref[1] += jnp.sum(y * y)


def _ln_prelu(y, stats_ref, lnw_ref, lnb_ref, a_ref, count):
    m = stats_ref[0] / count
    var = jnp.maximum(stats_ref[1] / count - m * m, 0.0)
    inv = 1.0 / (jnp.sqrt(var) + 1e-5)
    h = (y - m) * inv * lnw_ref[...] + lnb_ref[...]
    a = a_ref[0, 0]
    return jnp.where(h >= 0.0, h, a * h)


def _inv_degree(c0_ref, c1_ref):
    cnt = (c0_ref[...] + c1_ref[...])[:, 0:1]
    return 1.0 / jnp.maximum(cnt, 1.0)


def _layer0(s0a, s0b, c0, c1, x, Wl0, Wr0, bl0, lnw, lnb, a, Ws1):
    def body(s0a_ref, s0b_ref, c0_ref, c1_ref, x_ref, Wl_ref, Wr_ref,
             bl_ref, lnw_ref, lnb_ref, a_ref, Ws_ref,
             h1_ref, z0_ref, z1_ref, z2_ref, z3_ref, y_scr, stats_ref):
        i = pl.program_id(0)

        @pl.when(i < GRID)
        def _():
            invdeg = _inv_degree(c0_ref, c1_ref)
            agg = jnp.concatenate(
                [s0a_ref[...], s0b_ref[...]], axis=1) * invdeg
            y = (_dotT(agg, Wl_ref[...]) + _dotT(x_ref[...], Wr_ref[...])
                 + bl_ref[...])
            y_scr[pl.ds(i * R, R), :] = y
            _stats_add(stats_ref, y, i)

        @pl.when(i >= GRID)
        def _():
            ii = i - GRID
            y = y_scr[pl.ds(ii * R, R), :]
            h1 = _ln_prelu(y, stats_ref, lnw_ref, lnb_ref, a_ref,
                           float(N * DH))
            h1_ref[...] = h1
            z = h1 + _dotT(x_ref[...], Ws_ref[...])
            z0_ref[...] = z[:, 0:128]
            z1_ref[...] = z[:, 128:256]
            z2_ref[...] = z[:, 256:384]
            z3_ref[...] = z[:, 384:512]

    return pl.pallas_call(
        body,
        grid=(2 * GRID,),
        in_specs=[
            pl.BlockSpec((R, 128), _ph0),
            pl.BlockSpec((R, 128), _ph0),
            pl.BlockSpec((R, IDXW), _ph0),
            pl.BlockSpec((R, IDXW), _ph0),
            pl.BlockSpec((R, DIN), _both),
            pl.BlockSpec((DH, DIN), _const),
            pl.BlockSpec((DH, DIN), _const),
            pl.BlockSpec((1, DH), _const),
            pl.BlockSpec((1, DH), _const),
            pl.BlockSpec((1, DH), _const),
            _smem_spec(),
            pl.BlockSpec((DH, DIN), _const),
        ],
        out_specs=[pl.BlockSpec((R, DH), _ph1)]
        + [pl.BlockSpec((R, 128), _ph1)] * 4,
        out_shape=[jax.ShapeDtypeStruct((N, DH), jnp.float32)]
        + [jax.ShapeDtypeStruct((N, 128), jnp.float32)] * 4,
        scratch_shapes=[pltpu.VMEM((N, DH), jnp.float32),
                        pltpu.SMEM((2,), jnp.float32)],
    )(s0a, s0b, c0, c1, x, Wl0, Wr0, bl0, lnw, lnb, a, Ws1)


def _layer1(sums, c0, c1, z1c, h1, x, Wl1, Wr1, bl1, lnw, lnb, a, Ws2, Wl2):
    def body(sa_ref, sb_ref, sc_ref, sd_ref, c0_ref, c1_ref,
             za_ref, zb_ref, zc_ref, zd_ref, h1_ref, x_ref,
             Wl_ref, Wr_ref, bl_ref, lnw_ref, lnb_ref, a_ref,
             Ws_ref, Wl2_ref,
             z2_ref, u0_ref, u1_ref, y_scr, stats_ref):
        i = pl.program_id(0)

        @pl.when(i < GRID)
        def _():
            invdeg = _inv_degree(c0_ref, c1_ref)
            agg = jnp.concatenate(
                [sa_ref[...], sb_ref[...], sc_ref[...], sd_ref[...]],
                axis=1) * invdeg
            z1 = jnp.concatenate(
                [za_ref[...], zb_ref[...], zc_ref[...], zd_ref[...]], axis=1)
            y = (_dotT(agg, Wl_ref[...]) + _dotT(z1, Wr_ref[...])
                 + bl_ref[...])
            y_scr[pl.ds(i * R, R), :] = y
            _stats_add(stats_ref, y, i)

        @pl.when(i >= GRID)
        def _():
            ii = i - GRID
            y = y_scr[pl.ds(ii * R, R), :]
            h2 = _ln_prelu(y, stats_ref, lnw_ref, lnb_ref, a_ref,
                           float(N * DH))
            z2 = h1_ref[...] + h2 + _dotT(x_ref[...], Ws_ref[...])
            z2_ref[...] = z2
            u2 = _dotT(z2, Wl2_ref[...])
            u0_ref[...] = u2[:, 0:128]
            u1_ref[...] = u2[:, 128:256]

    return pl.pallas_call(
        body,
        grid=(2 * GRID,),
        in_specs=[pl.BlockSpec((R, 128), _ph0)] * 4
        + [pl.BlockSpec((R, IDXW), _ph0)] * 2
        + [pl.BlockSpec((R, 128), _ph0)] * 4
        + [pl.BlockSpec((R, DH), _ph1),
           pl.BlockSpec((R, DIN), _ph1),
           pl.BlockSpec((DH, DH), _const),
           pl.BlockSpec((DH, DH), _const),
           pl.BlockSpec((1, DH), _const),
           pl.BlockSpec((1, DH), _const),
           pl.BlockSpec((1, DH), _const),
           _smem_spec(),
           pl.BlockSpec((DH, DIN), _const),
           pl.BlockSpec((DOUT, DH), _const)],
        out_specs=[pl.BlockSpec((R, DH), _ph1),
                   pl.BlockSpec((R, 128), _ph1),
                   pl.BlockSpec((R, 128), _ph1)],
        out_shape=[jax.ShapeDtypeStruct((N, DH), jnp.float32),
                   jax.ShapeDtypeStruct((N, 128), jnp.float32),
                   jax.ShapeDtypeStruct((N, 128), jnp.float32)],
        scratch_shapes=[pltpu.VMEM((N, DH), jnp.float32),
                        pltpu.SMEM((2,), jnp.float32)],
    )(*sums, c0, c1, *z1c, h1, x, Wl1, Wr1, bl1, lnw, lnb, a, Ws2, Wl2)


def _layer2(s2a, s2b, c0, c1, z2, Wr2, bl2, lnw, lnb, a):
    def body(sa_ref, sb_ref, c0_ref, c1_ref, z2_ref, Wr_ref, bl_ref,
             lnw_ref, lnb_ref, a_ref, out_ref, y_scr, stats_ref):
        i = pl.program_id(0)

        @pl.when(i < GRID)
        def _():
            invdeg = _inv_degree(c0_ref, c1_ref)
            agg = jnp.concatenate(
                [sa_ref[...], sb_ref[...]], axis=1) * invdeg
            y = agg + _dotT(z2_ref[...], Wr_ref[...]) + bl_ref[...]
            y_scr[pl.ds(i * R, R), :] = y
            _stats_add(stats_ref, y, i)

        @pl.when(i >= GRID)
        def _():
            ii = i - GRID
            y = y_scr[pl.ds(ii * R, R), :]
            out_ref[...] = _ln_prelu(y, stats_ref, lnw_ref, lnb_ref,
                                     a_ref, float(N * DOUT))

    return pl.pallas_call(
        body,
        grid=(2 * GRID,),
        in_specs=[
            pl.BlockSpec((R, 128), _ph0),
            pl.BlockSpec((R, 128), _ph0),
            pl.BlockSpec((R, IDXW), _ph0),
            pl.BlockSpec((R, IDXW), _ph0),
            pl.BlockSpec((R, DH), _ph0),
            pl.BlockSpec((DOUT, DH), _const),
            pl.BlockSpec((1, DOUT), _const),
            pl.BlockSpec((1, DOUT), _const),
            pl.BlockSpec((1, DOUT), _const),
            _smem_spec(),
        ],
        out_specs=pl.BlockSpec((R, DOUT), _ph1),
        out_shape=jax.ShapeDtypeStruct((N, DOUT), jnp.float32),
        scratch_shapes=[pltpu.VMEM((N, DOUT), jnp.float32),
                        pltpu.SMEM((2,), jnp.float32)],
    )(s2a, s2b, c0, c1, z2, Wr2, bl2, lnw, lnb, a)


def kernel(x, edge_index, Wl0, bl0, Wr0, Wl1, bl1, Wr1, Wl2, bl2, Wr2,
           Ws1, Ws2, ln0_w, ln0_b, ln1_w, ln1_b, ln2_w, ln2_b, a0, a1, a2):
    src = edge_index[0]
    dst = edge_index[1]
    pad = EP - E
    srcp = jnp.concatenate(
        [src, jnp.zeros((pad,), jnp.int32)]).reshape(EP_ROWS, IDXW)
    dstp = jnp.concatenate(
        [dst, jnp.full((pad,), N, jnp.int32)]).reshape(EP_ROWS, IDXW)
    z128 = jnp.zeros((NPAD, 128), jnp.float32)
    ones128 = jnp.ones((IDXW, IDXW), jnp.float32)

    c0, c1 = _degree_counts(dstp, z128, ones128)

    x0 = x[:, :128]
    x1 = x[:, 128:]
    s0a, s0b = _segsum_pair(x0, x1, srcp, dstp, z128)
    h1, z1a, z1b, z1c, z1d = _layer0(
        s0a, s0b, c0, c1, x, Wl0, Wr0, bl0.reshape(1, -1),
        ln0_w.reshape(1, -1), ln0_b.reshape(1, -1),
        jnp.reshape(a0, (1, 1)), Ws1)

    s1a, s1b = _segsum_pair(z1a, z1b, srcp, dstp, z128)
    s1c, s1d = _segsum_pair(z1c, z1d, srcp, dstp, z128)
    z2, u2a, u2b = _layer1(
        (s1a, s1b, s1c, s1d), c0, c1, (z1a, z1b, z1c, z1d), h1, x,
        Wl1, Wr1, bl1.reshape(1, -1), ln1_w.reshape(1, -1),
        ln1_b.reshape(1, -1), jnp.reshape(a1, (1, 1)), Ws2, Wl2)

    s2a, s2b = _segsum_pair(u2a, u2b, srcp, dstp, z128)
    ret = _layer2(s2a, s2b, c0, c1, z2, Wr2, bl2.reshape(1, -1),
                  ln2_w.reshape(1, -1), ln2_b.reshape(1, -1),
                  jnp.reshape(a2, (1, 1)))
    return ret
